# baseline (speedup 1.0000x reference)
import jax
import jax.numpy as jnp
from jax.experimental import pallas as pl
from jax.experimental.pallas import tpu as pltpu

_HEADS = 32
_PAIRS = 16
_D = 128
_TB = 512
_TC = 16


def _fwht16(vs):
    for b in (8, 4, 2, 1):
        nv = [None] * _PAIRS
        for q in range(0, _PAIRS, 2 * b):
            for r in range(b):
                i0, i1 = q + r, q + r + b
                nv[i0] = vs[i0] + vs[i1]
                nv[i1] = vs[i0] - vs[i1]
        vs = nv
    return vs


def _compute_chunk(x_ref, w_ref, o_ref, u_ref):
    nchunks = _TB // _TC

    def mix1(i, carry):
        rows = pl.ds(i * _TC, _TC)
        vs = [
            jnp.concatenate(
                [x_ref[2 * p, rows, :], x_ref[2 * p + 1, rows, :]], axis=-1
            )
            for p in range(_PAIRS)
        ]
        vs = _fwht16(vs)
        for p in range(_PAIRS):
            u_ref[p, rows, :] = vs[p].astype(jnp.bfloat16)
        return carry

    jax.lax.fori_loop(0, nchunks, mix1, 0)

    for p in range(_PAIRS):
        z = jnp.dot(u_ref[p, :, :], w_ref[p], preferred_element_type=jnp.float32)
        o_ref[2 * p, :, :] = z[:, :_D]
        o_ref[2 * p + 1, :, :] = z[:, _D:]

    def mix2(i, carry):
        rows = pl.ds(i * _TC, _TC)
        vs = _fwht16(
            [
                jnp.concatenate(
                    [o_ref[2 * p, rows, :], o_ref[2 * p + 1, rows, :]], axis=-1
                )
                for p in range(_PAIRS)
            ]
        )
        for p in range(_PAIRS):
            o_ref[2 * p, rows, :] = vs[p][:, :_D]
            o_ref[2 * p + 1, rows, :] = vs[p][:, _D:]
        return carry

    jax.lax.fori_loop(0, nchunks, mix2, 0)


def _make_body(n_steps):
  def _body(x_hbm, w_ref, o_hbm, x_buf, o_buf, u_ref, in_sem, out_sem):
    b = pl.program_id(0)

    def dma_in(slot, step):
        pltpu.make_async_copy(
            x_hbm.at[b, :, pl.ds(step * _TB, _TB), :],
            x_buf.at[slot],
            in_sem.at[slot],
        ).start()

    def wait_in(slot):
        pltpu.make_async_copy(
            x_hbm.at[b, :, pl.ds(0, _TB), :], x_buf.at[slot], in_sem.at[slot]
        ).wait()

    def dma_out(slot, step):
        pltpu.make_async_copy(
            o_buf.at[slot],
            o_hbm.at[b, :, pl.ds(step * _TB, _TB), :],
            out_sem.at[slot],
        ).start()

    def wait_out(slot):
        pltpu.make_async_copy(
            o_buf.at[slot], o_hbm.at[b, :, pl.ds(0, _TB), :], out_sem.at[slot]
        ).wait()

    dma_in(0, 0)

    def loop(step, carry):
        cur = jax.lax.rem(step, 2)
        nxt = jax.lax.rem(step + 1, 2)

        @pl.when(step + 1 < n_steps)
        def _():
            dma_in(nxt, step + 1)

        wait_in(cur)

        @pl.when(step >= 2)
        def _():
            wait_out(cur)

        _compute_chunk(x_buf.at[cur], w_ref, o_buf.at[cur], u_ref)
        dma_out(cur, step)
        return carry

    jax.lax.fori_loop(0, n_steps, loop, 0)
    if n_steps >= 2:
        wait_out((n_steps - 2) % 2)
    wait_out((n_steps - 1) % 2)

  return _body


def kernel(x, W, beta):
    B, H, T, D = x.shape
    Wa, Wb = W[0::2], W[1::2]
    S, Dm = Wa + Wb, Wa - Wb
    top = jnp.concatenate([S, Dm], axis=-1)
    bot = jnp.concatenate([Dm, S], axis=-1)
    Wp = jnp.concatenate([top, bot], axis=-2)
    scale = jnp.concatenate([beta, beta]) * (1.0 / _HEADS)
    Wp = (Wp * scale[None, None, :]).astype(jnp.bfloat16)

    return pl.pallas_call(
        _make_body(T // _TB),
        grid=(B,),
        in_specs=[
            pl.BlockSpec(memory_space=pl.ANY),
            pl.BlockSpec((_PAIRS, 2 * D, 2 * D), lambda b: (0, 0, 0)),
        ],
        out_specs=pl.BlockSpec(memory_space=pl.ANY),
        out_shape=jax.ShapeDtypeStruct(x.shape, x.dtype),
        scratch_shapes=[
            pltpu.VMEM((2, H, _TB, D), jnp.float32),
            pltpu.VMEM((2, H, _TB, D), jnp.float32),
            pltpu.VMEM((_PAIRS, _TB, 2 * D), jnp.bfloat16),
            pltpu.SemaphoreType.DMA((2,)),
            pltpu.SemaphoreType.DMA((2,)),
        ],
        compiler_params=pltpu.CompilerParams(
            dimension_semantics=("parallel",),
        ),
    )(x, Wp)
